# baseline (speedup 1.0000x reference)
import functools

import numpy as np
import jax
import jax.numpy as jnp
from jax import lax
from jax.experimental import pallas as pl
from jax.experimental.pallas import tpu as pltpu
from jax.experimental.pallas import tpu_sc as plsc

D_MODEL = 2048
ROT_DIM = 64

_NC = 2
_NS = 16
_NW = _NC * _NS

_CHUNK = 16


def _emb_gather_body(table_hbm, idx_hbm, out_hbm, idx_v, rows_v, sem):
    wid = lax.axis_index("s") * _NC + lax.axis_index("c")
    nchunk = idx_v.shape[0]
    pltpu.sync_copy(idx_hbm.at[wid], idx_v)
    for c in range(nchunk):
        pltpu.async_copy(table_hbm.at[idx_v.at[c]], rows_v, sem).wait()
        pltpu.sync_copy(rows_v, out_hbm.at[pl.ds((wid * nchunk + c) * _CHUNK, _CHUNK)])


def _emb_gather(emb_table, idx_flat):
    rows = idx_flat.shape[0]
    r_per_w = rows // _NW
    nchunk = r_per_w // _CHUNK
    idx3 = idx_flat.reshape(_NW, nchunk, _CHUNK)
    mesh = plsc.VectorSubcoreMesh(core_axis_name="c", subcore_axis_name="s")
    k = functools.partial(
        pl.kernel,
        mesh=mesh,
        out_type=jax.ShapeDtypeStruct((rows, D_MODEL), jnp.float32),
        scratch_types=[
            pltpu.VMEM((nchunk, _CHUNK), jnp.int32),
            pltpu.VMEM((_CHUNK, D_MODEL), jnp.float32),
            pltpu.SemaphoreType.DMA,
        ],
    )(_emb_gather_body)
    return k(emb_table, idx3)


_BI = 256


def _mask_rot_body(theta_ref, am_row_ref, am_col_ref, mask_ref, rot_ref):
    b = pl.program_id(0)
    ib = pl.program_id(1)
    bi, s = mask_ref.shape[2], mask_ref.shape[3]

    pm_c = am_col_ref[0]
    pm_r = jnp.where(b == 0, am_row_ref[:, 0:1], am_row_ref[:, 1:2])
    row_ids = ib * bi + lax.broadcasted_iota(jnp.int32, (bi, s), 0)
    col_ids = lax.broadcasted_iota(jnp.int32, (bi, s), 1)
    mask = (pm_r == 1) & ((col_ids > row_ids) | (pm_c == 0))
    mask_ref[0, 0, :, :] = mask

    @pl.when((b == 0) & (ib == 0))
    def _():
        srot = rot_ref.shape[0]
        pos = lax.broadcasted_iota(jnp.int32, (srot, ROT_DIM), 0).astype(jnp.float32)
        angle = pos * theta_ref[...]
        parity = lax.broadcasted_iota(jnp.int32, (srot, ROT_DIM), 1) % 2
        rot_ref[...] = jnp.where(parity == 0, jnp.cos(angle), jnp.sin(angle))


def _mask_and_rotary(attention_mask, seq):
    bsz = attention_mask.shape[0]
    th = 1.0 / (10000.0 ** (np.arange(0, ROT_DIM, 2, dtype=np.float32) / ROT_DIM))
    theta = jnp.asarray(np.repeat(th, 2).reshape(1, ROT_DIM))
    am_row = attention_mask.T
    grid = (bsz, seq // _BI)
    mask, rot = pl.pallas_call(
        _mask_rot_body,
        grid=grid,
        in_specs=[
            pl.BlockSpec((1, ROT_DIM), lambda b, i: (0, 0)),
            pl.BlockSpec((_BI, bsz), lambda b, i: (i, 0)),
            pl.BlockSpec((1, 1, seq), lambda b, i: (b, 0, 0)),
        ],
        out_specs=[
            pl.BlockSpec((1, 1, _BI, seq), lambda b, i: (b, 0, i, 0)),
            pl.BlockSpec((seq, ROT_DIM), lambda b, i: (0, 0)),
        ],
        out_shape=[
            jax.ShapeDtypeStruct((bsz, 1, seq, seq), jnp.bool_),
            jax.ShapeDtypeStruct((seq, ROT_DIM), jnp.float32),
        ],
    )(theta, am_row, attention_mask.reshape(bsz, 1, seq))
    return mask, rot


def kernel(input_ids, attention_mask, emb_table):
    b, s = input_ids.shape
    idx_flat = input_ids.T.reshape(-1)
    hidden_flat = _emb_gather(emb_table, idx_flat)
    hidden_states = hidden_flat.reshape(s, b, D_MODEL)
    mask, rot = _mask_and_rotary(attention_mask, s)
    rotary_pos_emb = rot.reshape(1, s, ROT_DIM // 2, 2)
    return hidden_states, mask, rotary_pos_emb


# speedup vs baseline: 1.8647x; 1.8647x over previous
import functools

import numpy as np
import jax
import jax.numpy as jnp
from jax import lax
from jax.experimental import pallas as pl
from jax.experimental.pallas import tpu as pltpu
from jax.experimental.pallas import tpu_sc as plsc

D_MODEL = 2048
ROT_DIM = 64

_NC = 2
_NS = 16
_NW = _NC * _NS

_CHUNK = 16


def _emb_gather_body(table_hbm, idx_hbm, out_hbm, idx_v, rows_v, sem):
    wid = lax.axis_index("s") * _NC + lax.axis_index("c")
    nchunk = idx_v.shape[0]
    r_per_w = nchunk * _CHUNK
    b = wid // (_NW // 2)
    sblk = wid % (_NW // 2)
    pltpu.sync_copy(idx_hbm.at[wid], idx_v)
    for c in range(nchunk):
        pltpu.async_copy(table_hbm.at[idx_v.at[c]], rows_v, sem).wait()
        s0 = sblk * r_per_w + c * _CHUNK
        for dt in range(D_MODEL // 128):
            pltpu.sync_copy(
                rows_v.at[:, pl.ds(dt * 128, 128)],
                out_hbm.at[pl.ds(s0, _CHUNK), dt, b, :],
            )


def _emb_gather(emb_table, input_ids):
    bsz, seq = input_ids.shape
    rows = bsz * seq
    r_per_w = rows // _NW
    nchunk = r_per_w // _CHUNK
    idx3 = input_ids.reshape(_NW, nchunk, _CHUNK)
    mesh = plsc.VectorSubcoreMesh(core_axis_name="c", subcore_axis_name="s")
    k = functools.partial(
        pl.kernel,
        mesh=mesh,
        out_type=jax.ShapeDtypeStruct((seq, D_MODEL // 128, bsz, 128), jnp.float32),
        scratch_types=[
            pltpu.VMEM((nchunk, _CHUNK), jnp.int32),
            pltpu.VMEM((_CHUNK, D_MODEL), jnp.float32),
            pltpu.SemaphoreType.DMA,
        ],
    )(_emb_gather_body)
    out4 = k(emb_table, idx3)
    return out4.transpose(0, 2, 1, 3).reshape(seq, bsz, D_MODEL)


_BI = 256


def _mask_rot_body(theta_ref, am_row_ref, am_col_ref, mask_ref, rot_ref):
    b = pl.program_id(0)
    ib = pl.program_id(1)
    bi, s = mask_ref.shape[2], mask_ref.shape[3]

    pm_c = am_col_ref[0]
    pm_r = jnp.where(b == 0, am_row_ref[:, 0:1], am_row_ref[:, 1:2])
    row_ids = ib * bi + lax.broadcasted_iota(jnp.int32, (bi, s), 0)
    col_ids = lax.broadcasted_iota(jnp.int32, (bi, s), 1)
    mask = (pm_r == 1) & ((col_ids > row_ids) | (pm_c == 0))
    mask_ref[0, 0, :, :] = mask.astype(jnp.int8)

    @pl.when((b == 0) & (ib == 0))
    def _():
        srot = rot_ref.shape[0]
        pos = lax.broadcasted_iota(jnp.int32, (srot, ROT_DIM), 0).astype(jnp.float32)
        angle = pos * theta_ref[...]
        parity = lax.broadcasted_iota(jnp.int32, (srot, ROT_DIM), 1) % 2
        rot_ref[...] = jnp.where(parity == 0, jnp.cos(angle), jnp.sin(angle))


def _mask_and_rotary(attention_mask, seq):
    bsz = attention_mask.shape[0]
    th = 1.0 / (10000.0 ** (np.arange(0, ROT_DIM, 2, dtype=np.float32) / ROT_DIM))
    theta = jnp.asarray(np.repeat(th, 2).reshape(1, ROT_DIM))
    am_row = attention_mask.T
    grid = (bsz, seq // _BI)
    mask, rot = pl.pallas_call(
        _mask_rot_body,
        grid=grid,
        in_specs=[
            pl.BlockSpec((1, ROT_DIM), lambda b, i: (0, 0)),
            pl.BlockSpec((_BI, bsz), lambda b, i: (i, 0)),
            pl.BlockSpec((1, 1, seq), lambda b, i: (b, 0, 0)),
        ],
        out_specs=[
            pl.BlockSpec((1, 1, _BI, seq), lambda b, i: (b, 0, i, 0)),
            pl.BlockSpec((seq, ROT_DIM), lambda b, i: (0, 0)),
        ],
        out_shape=[
            jax.ShapeDtypeStruct((bsz, 1, seq, seq), jnp.int8),
            jax.ShapeDtypeStruct((seq, ROT_DIM), jnp.float32),
        ],
    )(theta, am_row, attention_mask.reshape(bsz, 1, seq))
    return mask.astype(jnp.bool_), rot


def kernel(input_ids, attention_mask, emb_table):
    b, s = input_ids.shape
    hidden_states = _emb_gather(emb_table, input_ids)
    mask, rot = _mask_and_rotary(attention_mask, s)
    rotary_pos_emb = rot.reshape(1, s, ROT_DIM // 2, 2)
    return hidden_states, mask, rotary_pos_emb


# speedup vs baseline: 2.0602x; 1.1048x over previous
import functools

import numpy as np
import jax
import jax.numpy as jnp
from jax import lax
from jax.experimental import pallas as pl
from jax.experimental.pallas import tpu as pltpu
from jax.experimental.pallas import tpu_sc as plsc

D_MODEL = 2048
ROT_DIM = 64

_NC = 2
_NS = 16
_NW = _NC * _NS

_CHUNK = 16


def _emb_gather_body(table_hbm, idx_hbm, out_hbm, idx_v, rows0, rows1, gsem0, gsem1, osem):
    wid = lax.axis_index("s") * _NC + lax.axis_index("c")
    nchunk = idx_v.shape[0]
    r_per_w = nchunk * _CHUNK
    b = wid // (_NW // 2)
    sblk = wid % (_NW // 2)
    pltpu.sync_copy(idx_hbm.at[wid], idx_v)
    bufs = (rows0, rows1)
    gsems = (gsem0, gsem1)
    pend_writes = [None, None]
    g = pltpu.async_copy(table_hbm.at[idx_v.at[0]], bufs[0], gsems[0])
    for c in range(nchunk):
        cur, nxt = c % 2, (c + 1) % 2
        if c + 1 < nchunk:
            if pend_writes[nxt] is not None:
                for cp in pend_writes[nxt]:
                    cp.wait()
            g_next = pltpu.async_copy(table_hbm.at[idx_v.at[c + 1]], bufs[nxt], gsems[nxt])
        g.wait()
        if c + 1 < nchunk:
            g = g_next
        s0 = sblk * r_per_w + c * _CHUNK
        wl = []
        for dt in range(D_MODEL // 128):
            wl.append(pltpu.async_copy(
                bufs[cur].at[:, pl.ds(dt * 128, 128)],
                out_hbm.at[pl.ds(s0, _CHUNK), dt, b, :],
                osem,
            ))
        pend_writes[cur] = wl
    for pw in pend_writes:
        if pw is not None:
            for cp in pw:
                cp.wait()


def _emb_gather(emb_table, input_ids):
    bsz, seq = input_ids.shape
    rows = bsz * seq
    r_per_w = rows // _NW
    nchunk = r_per_w // _CHUNK
    idx3 = input_ids.reshape(_NW, nchunk, _CHUNK)
    mesh = plsc.VectorSubcoreMesh(core_axis_name="c", subcore_axis_name="s")
    k = functools.partial(
        pl.kernel,
        mesh=mesh,
        out_type=jax.ShapeDtypeStruct((seq, D_MODEL // 128, bsz, 128), jnp.float32),
        scratch_types=[
            pltpu.VMEM((nchunk, _CHUNK), jnp.int32),
            pltpu.VMEM((_CHUNK, D_MODEL), jnp.float32),
            pltpu.VMEM((_CHUNK, D_MODEL), jnp.float32),
            pltpu.SemaphoreType.DMA,
            pltpu.SemaphoreType.DMA,
            pltpu.SemaphoreType.DMA,
        ],
    )(_emb_gather_body)
    out4 = k(emb_table, idx3)
    return out4.transpose(0, 2, 1, 3).reshape(seq, bsz, D_MODEL)


_BI = 256


def _mask_rot_body(theta_ref, am_row_ref, am_col_ref, mask_ref, rot_ref):
    ib = pl.program_id(0)
    bsz, _, bi, s = mask_ref.shape

    row_ids = ib * bi + lax.broadcasted_iota(jnp.int32, (bi, s), 0)
    col_ids = lax.broadcasted_iota(jnp.int32, (bi, s), 1)
    gt8 = (col_ids > row_ids).astype(jnp.int8)
    for b in range(bsz):
        npc8 = (am_col_ref[b] == 0).astype(jnp.int8)
        pr8 = (am_row_ref[:, b:b + 1] == 1).astype(jnp.int8)
        mask_ref[b, 0] = pr8 & (gt8 | npc8)

    @pl.when(ib == 0)
    def _():
        srot = rot_ref.shape[0]
        pos = lax.broadcasted_iota(jnp.int32, (srot, ROT_DIM), 0).astype(jnp.float32)
        angle = pos * theta_ref[...]
        parity = lax.broadcasted_iota(jnp.int32, (srot, ROT_DIM), 1) % 2
        rot_ref[...] = jnp.where(parity == 0, jnp.cos(angle), jnp.sin(angle))


def _mask_and_rotary(attention_mask, seq):
    bsz = attention_mask.shape[0]
    th = 1.0 / (10000.0 ** (np.arange(0, ROT_DIM, 2, dtype=np.float32) / ROT_DIM))
    theta = jnp.asarray(np.repeat(th, 2).reshape(1, ROT_DIM))
    am_row = attention_mask.T
    grid = (seq // _BI,)
    mask, rot = pl.pallas_call(
        _mask_rot_body,
        grid=grid,
        in_specs=[
            pl.BlockSpec((1, ROT_DIM), lambda i: (0, 0)),
            pl.BlockSpec((_BI, bsz), lambda i: (i, 0)),
            pl.BlockSpec((bsz, 1, seq), lambda i: (0, 0, 0)),
        ],
        out_specs=[
            pl.BlockSpec((bsz, 1, _BI, seq), lambda i: (0, 0, i, 0)),
            pl.BlockSpec((seq, ROT_DIM), lambda i: (0, 0)),
        ],
        out_shape=[
            jax.ShapeDtypeStruct((bsz, 1, seq, seq), jnp.int8),
            jax.ShapeDtypeStruct((seq, ROT_DIM), jnp.float32),
        ],
    )(theta, am_row, attention_mask.reshape(bsz, 1, seq))
    return mask.astype(jnp.bool_), rot


def kernel(input_ids, attention_mask, emb_table):
    b, s = input_ids.shape
    hidden_states = _emb_gather(emb_table, input_ids)
    mask, rot = _mask_and_rotary(attention_mask, s)
    rotary_pos_emb = rot.reshape(1, s, ROT_DIM // 2, 2)
    return hidden_states, mask, rotary_pos_emb


# speedup vs baseline: 2.1079x; 1.0232x over previous
import functools

import numpy as np
import jax
import jax.numpy as jnp
from jax import lax
from jax.experimental import pallas as pl
from jax.experimental.pallas import tpu as pltpu
from jax.experimental.pallas import tpu_sc as plsc

D_MODEL = 2048
ROT_DIM = 64

_NC = 2
_NS = 16
_NW = _NC * _NS

_CHUNK = 16


def _emb_gather_body(table_hbm, idx_hbm, out_hbm, idx_v, rows0, rows1, gsem0, gsem1, osem):
    wid = lax.axis_index("s") * _NC + lax.axis_index("c")
    nchunk = idx_v.shape[0]
    r_per_w = nchunk * _CHUNK
    b = wid // (_NW // 2)
    sblk = wid % (_NW // 2)
    pltpu.sync_copy(idx_hbm.at[wid], idx_v)
    bufs = (rows0, rows1)
    gsems = (gsem0, gsem1)
    pend_writes = [None, None]
    g = pltpu.async_copy(table_hbm.at[idx_v.at[0]], bufs[0], gsems[0])
    for c in range(nchunk):
        cur, nxt = c % 2, (c + 1) % 2
        if c + 1 < nchunk:
            if pend_writes[nxt] is not None:
                for cp in pend_writes[nxt]:
                    cp.wait()
            g_next = pltpu.async_copy(table_hbm.at[idx_v.at[c + 1]], bufs[nxt], gsems[nxt])
        g.wait()
        if c + 1 < nchunk:
            g = g_next
        s0 = sblk * r_per_w + c * _CHUNK
        wl = []
        for dt in range(D_MODEL // 128):
            wl.append(pltpu.async_copy(
                bufs[cur].at[:, pl.ds(dt * 128, 128)],
                out_hbm.at[pl.ds(s0, _CHUNK), dt, b, :],
                osem,
            ))
        pend_writes[cur] = wl
    for pw in pend_writes:
        if pw is not None:
            for cp in pw:
                cp.wait()


def _emb_gather(emb_table, input_ids):
    bsz, seq = input_ids.shape
    rows = bsz * seq
    r_per_w = rows // _NW
    nchunk = r_per_w // _CHUNK
    idx3 = input_ids.reshape(_NW, nchunk, _CHUNK)
    mesh = plsc.VectorSubcoreMesh(core_axis_name="c", subcore_axis_name="s")
    k = functools.partial(
        pl.kernel,
        mesh=mesh,
        out_type=jax.ShapeDtypeStruct((seq, D_MODEL // 128, bsz, 128), jnp.float32),
        scratch_types=[
            pltpu.VMEM((nchunk, _CHUNK), jnp.int32),
            pltpu.VMEM((_CHUNK, D_MODEL), jnp.float32),
            pltpu.VMEM((_CHUNK, D_MODEL), jnp.float32),
            pltpu.SemaphoreType.DMA,
            pltpu.SemaphoreType.DMA,
            pltpu.SemaphoreType.DMA,
        ],
    )(_emb_gather_body)
    out4 = k(emb_table, idx3)
    return out4.transpose(0, 2, 1, 3).reshape(seq, bsz, D_MODEL)


_BI = 256


def _mask_body(am_row_ref, am_col_ref, mask_ref):
    ib = pl.program_id(0)
    bsz, _, bi, s = mask_ref.shape

    row_ids = ib * bi + lax.broadcasted_iota(jnp.int32, (bi, s), 0)
    col_ids = lax.broadcasted_iota(jnp.int32, (bi, s), 1)
    gt8 = (col_ids > row_ids).astype(jnp.int8)
    for b in range(bsz):
        npc8 = (am_col_ref[b] == 0).astype(jnp.int8)
        pr8 = (am_row_ref[:, b:b + 1] == 1).astype(jnp.int8)
        mask_ref[b, 0] = pr8 & (gt8 | npc8)


def _rot_body(theta_ref, rot_ref):
    nk, nst, _, nsl = rot_ref.shape
    st = lax.broadcasted_iota(jnp.int32, (nk, nst, nsl), 1)
    sl = lax.broadcasted_iota(jnp.int32, (nk, nst, nsl), 2)
    pos = (st * nsl + sl).astype(jnp.float32)
    angle = pos * theta_ref[...][:, None, :]
    rot_ref[:, :, 0, :] = jnp.cos(angle)
    rot_ref[:, :, 1, :] = jnp.sin(angle)


def _mask_and_rotary(attention_mask, seq):
    bsz = attention_mask.shape[0]
    nk = ROT_DIM // 2
    th = 1.0 / (10000.0 ** (np.arange(0, ROT_DIM, 2, dtype=np.float32) / ROT_DIM))
    theta = jnp.asarray(np.repeat(th[:, None], 128, axis=1))
    am_row = attention_mask.T
    mask = pl.pallas_call(
        _mask_body,
        grid=(seq // _BI,),
        in_specs=[
            pl.BlockSpec((_BI, bsz), lambda i: (i, 0)),
            pl.BlockSpec((bsz, 1, seq), lambda i: (0, 0, 0)),
        ],
        out_specs=pl.BlockSpec((bsz, 1, _BI, seq), lambda i: (0, 0, i, 0)),
        out_shape=jax.ShapeDtypeStruct((bsz, 1, seq, seq), jnp.int8),
    )(am_row, attention_mask.reshape(bsz, 1, seq))
    rot4 = pl.pallas_call(
        _rot_body,
        out_shape=jax.ShapeDtypeStruct((nk, seq // 128, 2, 128), jnp.float32),
    )(theta)
    rot = rot4.transpose(1, 3, 0, 2).reshape(1, seq, nk, 2)
    return mask.astype(jnp.bool_), rot


def kernel(input_ids, attention_mask, emb_table):
    b, s = input_ids.shape
    hidden_states = _emb_gather(emb_table, input_ids)
    mask, rotary_pos_emb = _mask_and_rotary(attention_mask, s)
    return hidden_states, mask, rotary_pos_emb


# speedup vs baseline: 2.1507x; 1.0203x over previous
import functools

import numpy as np
import jax
import jax.numpy as jnp
from jax import lax
from jax.experimental import pallas as pl
from jax.experimental.pallas import tpu as pltpu
from jax.experimental.pallas import tpu_sc as plsc

D_MODEL = 2048
ROT_DIM = 64

_NC = 2
_NS = 16
_NW = _NC * _NS

_CHUNK = 16


_NBUF = 3


def _emb_gather_body(table_hbm, idx_hbm, out_hbm, idx_v,
                     rows0, rows1, rows2, gsem0, gsem1, gsem2, osem):
    wid = lax.axis_index("s") * _NC + lax.axis_index("c")
    r_per_w = idx_v.shape[0]
    nchunk = r_per_w // _CHUNK
    b = wid // (_NW // 2)
    sblk = wid % (_NW // 2)
    pltpu.sync_copy(idx_hbm.at[b, pl.ds(sblk * r_per_w, r_per_w)], idx_v)
    bufs = (rows0, rows1, rows2)
    gsems = (gsem0, gsem1, gsem2)
    pend_writes = [None] * _NBUF
    g = pltpu.async_copy(table_hbm.at[idx_v.at[pl.ds(0, _CHUNK)]], bufs[0], gsems[0])
    for c in range(nchunk):
        cur, nxt = c % _NBUF, (c + 1) % _NBUF
        if c + 1 < nchunk:
            if pend_writes[nxt] is not None:
                for cp in pend_writes[nxt]:
                    cp.wait()
                pend_writes[nxt] = None
            g_next = pltpu.async_copy(
                table_hbm.at[idx_v.at[pl.ds((c + 1) * _CHUNK, _CHUNK)]],
                bufs[nxt], gsems[nxt])
        g.wait()
        if c + 1 < nchunk:
            g = g_next
        s0 = sblk * r_per_w + c * _CHUNK
        wl = []
        for dt in range(D_MODEL // 128):
            wl.append(pltpu.async_copy(
                bufs[cur].at[:, pl.ds(dt * 128, 128)],
                out_hbm.at[pl.ds(s0, _CHUNK), dt, b, :],
                osem,
            ))
        pend_writes[cur] = wl
    for pw in pend_writes:
        if pw is not None:
            for cp in pw:
                cp.wait()


def _emb_gather(emb_table, input_ids):
    bsz, seq = input_ids.shape
    rows = bsz * seq
    r_per_w = rows // _NW
    mesh = plsc.VectorSubcoreMesh(core_axis_name="c", subcore_axis_name="s")
    k = functools.partial(
        pl.kernel,
        mesh=mesh,
        out_type=jax.ShapeDtypeStruct((seq, D_MODEL // 128, bsz, 128), jnp.float32),
        scratch_types=[
            pltpu.VMEM((r_per_w,), jnp.int32),
            pltpu.VMEM((_CHUNK, D_MODEL), jnp.float32),
            pltpu.VMEM((_CHUNK, D_MODEL), jnp.float32),
            pltpu.VMEM((_CHUNK, D_MODEL), jnp.float32),
            pltpu.SemaphoreType.DMA,
            pltpu.SemaphoreType.DMA,
            pltpu.SemaphoreType.DMA,
            pltpu.SemaphoreType.DMA,
        ],
    )(_emb_gather_body)
    out4 = k(emb_table, input_ids)
    return out4.transpose(0, 2, 1, 3).reshape(seq, bsz, D_MODEL)


_BI = 256


def _mask_body(am_row_ref, am_col_ref, mask_ref):
    ib = pl.program_id(0)
    bsz, _, bi, s = mask_ref.shape

    row_ids = ib * bi + lax.broadcasted_iota(jnp.int32, (bi, s), 0)
    col_ids = lax.broadcasted_iota(jnp.int32, (bi, s), 1)
    gt8 = (col_ids > row_ids).astype(jnp.int8)
    for b in range(bsz):
        npc8 = (am_col_ref[b] == 0).astype(jnp.int8)
        pr8 = (am_row_ref[:, b:b + 1] == 1).astype(jnp.int8)
        mask_ref[b, 0] = pr8 & (gt8 | npc8)


def _rot_body(theta_ref, rot_ref):
    nk, nst, _, nsl = rot_ref.shape
    st = lax.broadcasted_iota(jnp.int32, (nk, nst, nsl), 1)
    sl = lax.broadcasted_iota(jnp.int32, (nk, nst, nsl), 2)
    pos = (st * nsl + sl).astype(jnp.float32)
    angle = pos * theta_ref[...][:, None, :]
    rot_ref[:, :, 0, :] = jnp.cos(angle)
    rot_ref[:, :, 1, :] = jnp.sin(angle)


def _mask_and_rotary(attention_mask, seq):
    bsz = attention_mask.shape[0]
    nk = ROT_DIM // 2
    th = 1.0 / (10000.0 ** (np.arange(0, ROT_DIM, 2, dtype=np.float32) / ROT_DIM))
    theta = jnp.asarray(np.repeat(th[:, None], 128, axis=1))
    am_row = attention_mask.T
    mask = pl.pallas_call(
        _mask_body,
        grid=(seq // _BI,),
        in_specs=[
            pl.BlockSpec((_BI, bsz), lambda i: (i, 0)),
            pl.BlockSpec((bsz, 1, seq), lambda i: (0, 0, 0)),
        ],
        out_specs=pl.BlockSpec((bsz, 1, _BI, seq), lambda i: (0, 0, i, 0)),
        out_shape=jax.ShapeDtypeStruct((bsz, 1, seq, seq), jnp.int8),
    )(am_row, attention_mask.reshape(bsz, 1, seq))
    rot4 = pl.pallas_call(
        _rot_body,
        out_shape=jax.ShapeDtypeStruct((nk, seq // 128, 2, 128), jnp.float32),
    )(theta)
    rot = rot4.transpose(1, 3, 0, 2).reshape(1, seq, nk, 2)
    return mask.astype(jnp.bool_), rot


def kernel(input_ids, attention_mask, emb_table):
    b, s = input_ids.shape
    hidden_states = _emb_gather(emb_table, input_ids)
    mask, rotary_pos_emb = _mask_and_rotary(attention_mask, s)
    return hidden_states, mask, rotary_pos_emb


# speedup vs baseline: 2.2158x; 1.0302x over previous
import functools

import numpy as np
import jax
import jax.numpy as jnp
from jax import lax
from jax.experimental import pallas as pl
from jax.experimental.pallas import tpu as pltpu
from jax.experimental.pallas import tpu_sc as plsc

D_MODEL = 2048
ROT_DIM = 64

_NC = 2
_NS = 16
_NW = _NC * _NS

_CHUNK = 16


_NBUF = 3


def _emb_gather_body(table_hbm, idx_hbm, out_hbm, idx_v,
                     rows0, rows1, rows2, gsem0, gsem1, gsem2,
                     osem0, osem1, osem2):
    wid = lax.axis_index("s") * _NC + lax.axis_index("c")
    r_per_w = idx_v.shape[0]
    nchunk = r_per_w // _CHUNK
    b = wid // (_NW // 2)
    sblk = wid % (_NW // 2)
    pltpu.sync_copy(idx_hbm.at[b, pl.ds(sblk * r_per_w, r_per_w)], idx_v)
    bufs = (rows0, rows1, rows2)
    gsems = (gsem0, gsem1, gsem2)
    osems = (osem0, osem1, osem2)
    ndt = D_MODEL // 128
    has_writes = [False] * _NBUF
    g = pltpu.async_copy(table_hbm.at[idx_v.at[pl.ds(0, _CHUNK)]], bufs[0], gsems[0])
    for c in range(nchunk):
        cur, nxt = c % _NBUF, (c + 1) % _NBUF
        if c + 1 < nchunk:
            if has_writes[nxt]:
                pltpu.make_async_copy(
                    table_hbm.at[pl.ds(0, _CHUNK)], bufs[nxt], osems[nxt]).wait()
                has_writes[nxt] = False
            g_next = pltpu.async_copy(
                table_hbm.at[idx_v.at[pl.ds((c + 1) * _CHUNK, _CHUNK)]],
                bufs[nxt], gsems[nxt])
        g.wait()
        if c + 1 < nchunk:
            g = g_next
        s0 = sblk * r_per_w + c * _CHUNK
        buf_c, osem_c = bufs[cur], osems[cur]

        def _write(dt, _, buf=buf_c, sem=osem_c, s0=s0, b=b):
            pltpu.async_copy(
                buf.at[:, pl.ds(dt * 128, 128)],
                out_hbm.at[pl.ds(s0, _CHUNK), dt, b, :],
                sem,
            )
            return _

        lax.fori_loop(0, ndt, _write, 0)
        has_writes[cur] = True
    for r in range(_NBUF):
        if has_writes[r]:
            pltpu.make_async_copy(
                table_hbm.at[pl.ds(0, _CHUNK)], bufs[r], osems[r]).wait()


def _emb_gather(emb_table, input_ids):
    bsz, seq = input_ids.shape
    rows = bsz * seq
    r_per_w = rows // _NW
    mesh = plsc.VectorSubcoreMesh(core_axis_name="c", subcore_axis_name="s")
    k = functools.partial(
        pl.kernel,
        mesh=mesh,
        out_type=jax.ShapeDtypeStruct((seq, D_MODEL // 128, bsz, 128), jnp.float32),
        scratch_types=[
            pltpu.VMEM((r_per_w,), jnp.int32),
            pltpu.VMEM((_CHUNK, D_MODEL), jnp.float32),
            pltpu.VMEM((_CHUNK, D_MODEL), jnp.float32),
            pltpu.VMEM((_CHUNK, D_MODEL), jnp.float32),
            pltpu.SemaphoreType.DMA,
            pltpu.SemaphoreType.DMA,
            pltpu.SemaphoreType.DMA,
            pltpu.SemaphoreType.DMA,
            pltpu.SemaphoreType.DMA,
            pltpu.SemaphoreType.DMA,
        ],
    )(_emb_gather_body)
    out4 = k(emb_table, input_ids)
    return out4.transpose(0, 2, 1, 3).reshape(seq, bsz, D_MODEL)


_BI = 256


def _mask_body(am_row_ref, am_col_ref, mask_ref):
    ib = pl.program_id(0)
    bsz, _, bi, s = mask_ref.shape

    row_ids = ib * bi + lax.broadcasted_iota(jnp.int32, (bi, s), 0)
    col_ids = lax.broadcasted_iota(jnp.int32, (bi, s), 1)
    gt8 = (col_ids > row_ids).astype(jnp.int8)
    for b in range(bsz):
        npc8 = (am_col_ref[b] == 0).astype(jnp.int8)
        pr8 = (am_row_ref[:, b:b + 1] == 1).astype(jnp.int8)
        mask_ref[b, 0] = pr8 & (gt8 | npc8)


def _rot_body(theta_ref, rot_ref):
    nk, nst, _, nsl = rot_ref.shape
    st = lax.broadcasted_iota(jnp.int32, (nk, nst, nsl), 1)
    sl = lax.broadcasted_iota(jnp.int32, (nk, nst, nsl), 2)
    pos = (st * nsl + sl).astype(jnp.float32)
    angle = pos * theta_ref[...][:, None, :]
    rot_ref[:, :, 0, :] = jnp.cos(angle)
    rot_ref[:, :, 1, :] = jnp.sin(angle)


def _mask_and_rotary(attention_mask, seq):
    bsz = attention_mask.shape[0]
    nk = ROT_DIM // 2
    th = 1.0 / (10000.0 ** (np.arange(0, ROT_DIM, 2, dtype=np.float32) / ROT_DIM))
    theta = jnp.asarray(np.repeat(th[:, None], 128, axis=1))
    am_row = attention_mask.T
    mask = pl.pallas_call(
        _mask_body,
        grid=(seq // _BI,),
        in_specs=[
            pl.BlockSpec((_BI, bsz), lambda i: (i, 0)),
            pl.BlockSpec((bsz, 1, seq), lambda i: (0, 0, 0)),
        ],
        out_specs=pl.BlockSpec((bsz, 1, _BI, seq), lambda i: (0, 0, i, 0)),
        out_shape=jax.ShapeDtypeStruct((bsz, 1, seq, seq), jnp.int8),
    )(am_row, attention_mask.reshape(bsz, 1, seq))
    rot4 = pl.pallas_call(
        _rot_body,
        out_shape=jax.ShapeDtypeStruct((nk, seq // 128, 2, 128), jnp.float32),
    )(theta)
    rot = rot4.transpose(1, 3, 0, 2).reshape(1, seq, nk, 2)
    return mask.astype(jnp.bool_), rot


def kernel(input_ids, attention_mask, emb_table):
    b, s = input_ids.shape
    hidden_states = _emb_gather(emb_table, input_ids)
    mask, rotary_pos_emb = _mask_and_rotary(attention_mask, s)
    return hidden_states, mask, rotary_pos_emb
